# baseline (speedup 1.0000x reference)
import functools

import jax
import jax.numpy as jnp
from jax import lax
from jax.experimental import pallas as pl
from jax.experimental.pallas import tpu as pltpu
from jax.experimental.pallas import tpu_sc as plsc

B, L, V, H, E = 4096, 200, 1000000, 64, 32

_NC, _NS = 2, 16
_NW = _NC * _NS
_BPW = B // _NW
_LP = 256
_HP = 128


def _acc_row(buf):
    zero = jnp.zeros((16,), jnp.float32)

    @pl.loop(0, L, init_carry=(zero, zero, zero, zero), unroll=8)
    def acc(j, carry):
        a0, a1, a2, a3 = carry
        a0 = a0 + buf[j, pl.ds(0, 16)]
        a1 = a1 + buf[j, pl.ds(16, 16)]
        a2 = a2 + buf[j, pl.ds(32, 16)]
        a3 = a3 + buf[j, pl.ds(48, 16)]
        return a0, a1, a2, a3

    return acc


def _sc_pool(x, table):
    mesh = plsc.VectorSubcoreMesh(core_axis_name="c", subcore_axis_name="s")
    inv_l = jnp.float32(1.0 / L)

    @functools.partial(
        pl.kernel,
        out_type=jax.ShapeDtypeStruct((B, H), jnp.float32),
        mesh=mesh,
        scratch_types=[
            pltpu.VMEM((_BPW, _LP), jnp.int32),
            pltpu.VMEM((L, _HP), jnp.float32),
            pltpu.VMEM((L, _HP), jnp.float32),
            pltpu.VMEM((_BPW, H), jnp.float32),
            pltpu.SemaphoreType.DMA,
            pltpu.SemaphoreType.DMA,
        ],
    )
    def pool(x_hbm, tbl_hbm, out_hbm, idx_v, buf0, buf1, out_v, sem0, sem1):
        wid = lax.axis_index("s") * _NC + lax.axis_index("c")

        pltpu.sync_copy(x_hbm.at[pl.ds(wid * _BPW, _BPW)], idx_v)

        def fire(row, buf, sem):
            pltpu.async_copy(
                tbl_hbm.at[idx_v.at[row, pl.ds(0, 128)]],
                buf.at[pl.ds(0, 128)], sem)
            pltpu.async_copy(
                tbl_hbm.at[idx_v.at[row, pl.ds(128, L - 128)]],
                buf.at[pl.ds(128, L - 128)], sem)

        def drain(buf, sem):
            pltpu.make_async_copy(tbl_hbm.at[pl.ds(0, L)], buf, sem).wait()

        def acc_store(row, buf):
            a0, a1, a2, a3 = _acc_row(buf)
            out_v[row, pl.ds(0, 16)] = a0 * inv_l
            out_v[row, pl.ds(16, 16)] = a1 * inv_l
            out_v[row, pl.ds(32, 16)] = a2 * inv_l
            out_v[row, pl.ds(48, 16)] = a3 * inv_l

        fire(0, buf0, sem0)
        fire(1, buf1, sem1)

        @pl.loop(0, _BPW - 2, step=2)
        def steady(b):
            drain(buf0, sem0)
            acc_store(b, buf0)
            fire(b + 2, buf0, sem0)
            drain(buf1, sem1)
            acc_store(b + 1, buf1)
            fire(b + 3, buf1, sem1)

        drain(buf0, sem0)
        acc_store(_BPW - 2, buf0)
        drain(buf1, sem1)
        acc_store(_BPW - 1, buf1)

        pltpu.sync_copy(out_v, out_hbm.at[pl.ds(wid * _BPW, _BPW)])

    return pool(x, table)


def _mlp_body(pooled_ref, w1_ref, b1_ref, gamma_ref, beta_ref, wout_ref,
              bout_ref, out_ref):
    h = jnp.dot(pooled_ref[...], w1_ref[...],
                preferred_element_type=jnp.float32) + b1_ref[...]
    mu = jnp.mean(h, axis=0, keepdims=True)
    var = jnp.mean((h - mu) * (h - mu), axis=0, keepdims=True)
    hn = (h - mu) * lax.rsqrt(var + 1e-5)
    hn = hn * gamma_ref[...] + beta_ref[...]
    hn = jnp.maximum(hn, 0.0)
    out_ref[...] = jnp.dot(hn, wout_ref[...],
                           preferred_element_type=jnp.float32) + bout_ref[...]


def kernel(x, table, W1, b1, gamma, beta, Wout, bout):
    xp = jnp.pad(x.astype(jnp.int32), ((0, 0), (0, _LP - L)))
    eye = jnp.eye(H, _HP, dtype=jnp.bfloat16)
    th = table.astype(jnp.bfloat16)
    tl = (table - th.astype(jnp.float32)).astype(jnp.bfloat16)
    tp = (jnp.dot(th, eye, preferred_element_type=jnp.float32)
          + jnp.dot(tl, eye, preferred_element_type=jnp.float32))
    pooled = _sc_pool(xp, tp)
    return pl.pallas_call(
        _mlp_body,
        out_shape=jax.ShapeDtypeStruct((B, E), jnp.float32),
    )(pooled, W1, b1.reshape(1, H), gamma.reshape(1, H),
      beta.reshape(1, H), Wout, bout.reshape(1, E))


# speedup vs baseline: 1.8521x; 1.8521x over previous
import functools

import jax
import jax.numpy as jnp
from jax import lax
from jax.experimental import pallas as pl
from jax.experimental.pallas import tpu as pltpu
from jax.experimental.pallas import tpu_sc as plsc

B, L, V, H, E = 4096, 200, 1000000, 64, 32

_NC, _NS = 2, 16
_NW = _NC * _NS
_BPW = B // _NW
_LP = 256
_HP = 128


def _acc_row(buf):
    zero = jnp.zeros((16,), jnp.float32)

    @pl.loop(0, L, init_carry=(zero, zero, zero, zero), unroll=8)
    def acc(j, carry):
        a0, a1, a2, a3 = carry
        a0 = a0 + buf[j, pl.ds(0, 16)]
        a1 = a1 + buf[j, pl.ds(16, 16)]
        a2 = a2 + buf[j, pl.ds(32, 16)]
        a3 = a3 + buf[j, pl.ds(48, 16)]
        return a0, a1, a2, a3

    return acc


def _sc_pool(x, table):
    mesh = plsc.VectorSubcoreMesh(core_axis_name="c", subcore_axis_name="s")
    inv_l = jnp.float32(1.0 / L)

    @functools.partial(
        pl.kernel,
        out_type=jax.ShapeDtypeStruct((B, H), jnp.float32),
        mesh=mesh,
        scratch_types=[
            pltpu.VMEM((_BPW, _LP), jnp.int32),
            pltpu.VMEM((L, _HP), jnp.float32),
            pltpu.VMEM((L, _HP), jnp.float32),
            pltpu.VMEM((_BPW, H), jnp.float32),
            pltpu.SemaphoreType.DMA,
            pltpu.SemaphoreType.DMA,
        ],
    )
    def pool(x_hbm, tbl_hbm, out_hbm, idx_v, buf0, buf1, out_v, sem0, sem1):
        wid = lax.axis_index("s") * _NC + lax.axis_index("c")

        pltpu.sync_copy(x_hbm.at[pl.ds(wid * _BPW, _BPW)], idx_v)

        def fire(row, buf, sem):
            pltpu.async_copy(
                tbl_hbm.at[idx_v.at[row, pl.ds(0, 128)]],
                buf.at[pl.ds(0, 128)], sem)
            pltpu.async_copy(
                tbl_hbm.at[idx_v.at[row, pl.ds(128, L - 128)]],
                buf.at[pl.ds(128, L - 128)], sem)

        def drain(buf, sem):
            pltpu.make_async_copy(tbl_hbm.at[pl.ds(0, L)], buf, sem).wait()

        def acc_store(row, buf):
            a0, a1, a2, a3 = _acc_row(buf)
            out_v[row, pl.ds(0, 16)] = a0 * inv_l
            out_v[row, pl.ds(16, 16)] = a1 * inv_l
            out_v[row, pl.ds(32, 16)] = a2 * inv_l
            out_v[row, pl.ds(48, 16)] = a3 * inv_l

        fire(0, buf0, sem0)
        fire(1, buf1, sem1)

        @pl.loop(0, _BPW - 2, step=2)
        def steady(b):
            drain(buf0, sem0)
            acc_store(b, buf0)
            fire(b + 2, buf0, sem0)
            drain(buf1, sem1)
            acc_store(b + 1, buf1)
            fire(b + 3, buf1, sem1)

        drain(buf0, sem0)
        acc_store(_BPW - 2, buf0)
        drain(buf1, sem1)
        acc_store(_BPW - 1, buf1)

        pltpu.sync_copy(out_v, out_hbm.at[pl.ds(wid * _BPW, _BPW)])

    return pool(x, table)


def _mlp_body(pooled_ref, w1_ref, b1_ref, gamma_ref, beta_ref, wout_ref,
              bout_ref, out_ref):
    h = jnp.dot(pooled_ref[...], w1_ref[...],
                preferred_element_type=jnp.float32) + b1_ref[...]
    mu = jnp.mean(h, axis=0, keepdims=True)
    var = jnp.mean((h - mu) * (h - mu), axis=0, keepdims=True)
    hn = (h - mu) * lax.rsqrt(var + 1e-5)
    hn = hn * gamma_ref[...] + beta_ref[...]
    hn = jnp.maximum(hn, 0.0)
    out_ref[...] = jnp.dot(hn, wout_ref[...],
                           preferred_element_type=jnp.float32) + bout_ref[...]


def kernel(x, table, W1, b1, gamma, beta, Wout, bout):
    xp = jnp.pad(x.astype(jnp.int32), ((0, 0), (0, _LP - L)))
    tp = jnp.dot(table, jnp.eye(H, _HP, dtype=jnp.float32),
                 preferred_element_type=jnp.float32)
    pooled = _sc_pool(xp, tp)
    return pl.pallas_call(
        _mlp_body,
        out_shape=jax.ShapeDtypeStruct((B, E), jnp.float32),
    )(pooled, W1, b1.reshape(1, H), gamma.reshape(1, H),
      beta.reshape(1, H), Wout, bout.reshape(1, E))


# speedup vs baseline: 1.9461x; 1.0508x over previous
import functools

import jax
import jax.numpy as jnp
from jax import lax
from jax.experimental import pallas as pl
from jax.experimental.pallas import tpu as pltpu
from jax.experimental.pallas import tpu_sc as plsc

B, L, V, H, E = 4096, 200, 1000000, 64, 32

_NC, _NS = 2, 16
_NW = _NC * _NS
_BPW = B // _NW
_LP = 256
_HP = 128


def _acc_row(buf):
    zero = jnp.zeros((16,), jnp.float32)

    @pl.loop(0, L, init_carry=(zero, zero, zero, zero), unroll=8)
    def acc(j, carry):
        a0, a1, a2, a3 = carry
        a0 = a0 + buf[j, pl.ds(0, 16)]
        a1 = a1 + buf[j, pl.ds(16, 16)]
        a2 = a2 + buf[j, pl.ds(32, 16)]
        a3 = a3 + buf[j, pl.ds(48, 16)]
        return a0, a1, a2, a3

    return acc


def _sc_pool(x, table):
    mesh = plsc.VectorSubcoreMesh(core_axis_name="c", subcore_axis_name="s")
    inv_l = jnp.float32(1.0 / L)

    @functools.partial(
        pl.kernel,
        out_type=jax.ShapeDtypeStruct((B, H), jnp.float32),
        mesh=mesh,
        scratch_types=[
            pltpu.VMEM((_BPW, _LP), jnp.int32),
            pltpu.VMEM((L, _HP), jnp.float32),
            pltpu.VMEM((L, _HP), jnp.float32),
            pltpu.VMEM((L, _HP), jnp.float32),
            pltpu.VMEM((_BPW, H), jnp.float32),
            pltpu.SemaphoreType.DMA,
            pltpu.SemaphoreType.DMA,
            pltpu.SemaphoreType.DMA,
        ],
    )
    def pool(x_hbm, tbl_hbm, out_hbm, idx_v, buf0, buf1, buf2, out_v,
             sem0, sem1, sem2):
        wid = lax.axis_index("s") * _NC + lax.axis_index("c")

        pltpu.sync_copy(x_hbm.at[pl.ds(wid * _BPW, _BPW)], idx_v)

        def fire(row, buf, sem):
            pltpu.async_copy(
                tbl_hbm.at[idx_v.at[row, pl.ds(0, 128)]],
                buf.at[pl.ds(0, 128)], sem)
            pltpu.async_copy(
                tbl_hbm.at[idx_v.at[row, pl.ds(128, L - 128)]],
                buf.at[pl.ds(128, L - 128)], sem)

        def drain(buf, sem):
            pltpu.make_async_copy(tbl_hbm.at[pl.ds(0, L)], buf, sem).wait()

        def acc_store(row, buf):
            a0, a1, a2, a3 = _acc_row(buf)
            out_v[row, pl.ds(0, 16)] = a0 * inv_l
            out_v[row, pl.ds(16, 16)] = a1 * inv_l
            out_v[row, pl.ds(32, 16)] = a2 * inv_l
            out_v[row, pl.ds(48, 16)] = a3 * inv_l

        fire(0, buf0, sem0)
        fire(1, buf1, sem1)
        fire(2, buf2, sem2)

        @pl.loop(0, _BPW - 2, step=3)
        def steady(b):
            drain(buf0, sem0)
            acc_store(b, buf0)
            fire(b + 3, buf0, sem0)
            drain(buf1, sem1)
            acc_store(b + 1, buf1)
            fire(b + 4, buf1, sem1)
            drain(buf2, sem2)
            acc_store(b + 2, buf2)

            @pl.when(b + 5 < _BPW)
            def _():
                fire(b + 5, buf2, sem2)

        drain(buf0, sem0)
        acc_store(_BPW - 2, buf0)
        drain(buf1, sem1)
        acc_store(_BPW - 1, buf1)

        pltpu.sync_copy(out_v, out_hbm.at[pl.ds(wid * _BPW, _BPW)])

    return pool(x, table)


def _mlp_body(pooled_ref, w1_ref, b1_ref, gamma_ref, beta_ref, wout_ref,
              bout_ref, out_ref):
    h = jnp.dot(pooled_ref[...], w1_ref[...],
                preferred_element_type=jnp.float32) + b1_ref[...]
    mu = jnp.mean(h, axis=0, keepdims=True)
    var = jnp.mean((h - mu) * (h - mu), axis=0, keepdims=True)
    hn = (h - mu) * lax.rsqrt(var + 1e-5)
    hn = hn * gamma_ref[...] + beta_ref[...]
    hn = jnp.maximum(hn, 0.0)
    out_ref[...] = jnp.dot(hn, wout_ref[...],
                           preferred_element_type=jnp.float32) + bout_ref[...]


def kernel(x, table, W1, b1, gamma, beta, Wout, bout):
    xp = jnp.pad(x.astype(jnp.int32), ((0, 0), (0, _LP - L)))
    tp = jnp.dot(table, jnp.eye(H, _HP, dtype=jnp.float32),
                 preferred_element_type=jnp.float32)
    pooled = _sc_pool(xp, tp)
    return pl.pallas_call(
        _mlp_body,
        out_shape=jax.ShapeDtypeStruct((B, E), jnp.float32),
    )(pooled, W1, b1.reshape(1, H), gamma.reshape(1, H),
      beta.reshape(1, H), Wout, bout.reshape(1, E))
